# baseline (speedup 1.0000x reference)
import functools

import jax
import jax.numpy as jnp
from jax.experimental import pallas as pl
from jax.experimental.pallas import tpu as pltpu

TEMPERATURE = 1.0
EPS = 1e-20
B, V = 128, 100000

ROWS = 8
NROW = B // ROWS
HOT_W = 4352

INT_MAX = 2**31 - 1


def _gumbel_argmax_kernel(l_ref, u_ref, hot_ref, idx_ref):
    r = pl.program_id(0)

    @pl.when(r < NROW)
    def _argmax():
        g = -jnp.log(-jnp.log(u_ref[...] + EPS) + EPS)
        s = l_ref[...] + g
        bmax = jnp.max(s, axis=1, keepdims=True)
        col = jax.lax.broadcasted_iota(jnp.int32, s.shape, 1)
        bidx = jnp.min(jnp.where(s == bmax, col, INT_MAX), axis=1,
                       keepdims=True)
        idx_ref[pl.ds(r * ROWS, ROWS), :] = bidx

    @pl.when(r == NROW)
    def _one_hot():
        idx = idx_ref[...]
        base = 0
        while base < V:
            w = min(HOT_W, V - base)
            col = (jax.lax.broadcasted_iota(jnp.int32, (B, w), 1) + base)
            anyhot = jnp.any(col == idx, axis=0, keepdims=True)
            hot_ref[:, pl.ds(base, w)] = anyhot.astype(jnp.float32)
            base += w


@functools.partial(jax.jit, static_argnames=("interpret",))
def kernel(logits, gumbel_u, interpret=False):
    hot = pl.pallas_call(
        _gumbel_argmax_kernel,
        grid=(NROW + 1,),
        in_specs=[
            pl.BlockSpec((ROWS, V), lambda r: (jnp.minimum(r, NROW - 1), 0)),
            pl.BlockSpec((ROWS, V), lambda r: (jnp.minimum(r, NROW - 1), 0)),
        ],
        out_specs=pl.BlockSpec((1, V), lambda r: (0, 0)),
        out_shape=jax.ShapeDtypeStruct((1, V), jnp.float32),
        scratch_shapes=[pltpu.VMEM((B, 1), jnp.int32)],
        interpret=interpret,
    )(logits, gumbel_u)

    return jax.lax.pad(hot, jnp.float32(0.0), ((0, B - 1, 0), (0, 0, 0)))


# speedup vs baseline: 1.0941x; 1.0941x over previous
import functools

import jax
import jax.numpy as jnp
from jax.experimental import pallas as pl
from jax.experimental.pallas import tpu as pltpu

TEMPERATURE = 1.0
EPS = 1e-20
B, V = 128, 100000

ROWS = 8
NROW = B // ROWS
NBUF = 4
HOT_W = 4352

INT_MAX = 2**31 - 1


def _gumbel_argmax_kernel(l_hbm, u_hbm, hot_ref, lbuf, ubuf, idx_ref,
                          lsem, usem):
    def issue(g):
        slot = g % NBUF
        pltpu.make_async_copy(l_hbm.at[pl.ds(g * ROWS, ROWS), :],
                              lbuf.at[slot], lsem.at[slot]).start()
        pltpu.make_async_copy(u_hbm.at[pl.ds(g * ROWS, ROWS), :],
                              ubuf.at[slot], usem.at[slot]).start()

    for g in range(min(NBUF, NROW)):
        issue(g)

    for g in range(NROW):
        slot = g % NBUF
        pltpu.make_async_copy(l_hbm.at[pl.ds(g * ROWS, ROWS), :],
                              lbuf.at[slot], lsem.at[slot]).wait()
        pltpu.make_async_copy(u_hbm.at[pl.ds(g * ROWS, ROWS), :],
                              ubuf.at[slot], usem.at[slot]).wait()
        gum = -jnp.log(-jnp.log(ubuf[slot] + EPS) + EPS)
        s = lbuf[slot] + gum
        bmax = jnp.max(s, axis=1, keepdims=True)
        col = jax.lax.broadcasted_iota(jnp.int32, s.shape, 1)
        idx_ref[pl.ds(g * ROWS, ROWS), :] = jnp.min(
            jnp.where(s == bmax, col, INT_MAX), axis=1, keepdims=True)
        if g + NBUF < NROW:
            issue(g + NBUF)

    idx = idx_ref[...]
    base = 0
    while base < V:
        w = min(HOT_W, V - base)
        col = jax.lax.broadcasted_iota(jnp.int32, (B, w), 1) + base
        anyhot = jnp.any(col == idx, axis=0, keepdims=True)
        hot_ref[:, pl.ds(base, w)] = anyhot.astype(jnp.float32)
        base += w


@functools.partial(jax.jit, static_argnames=("interpret",))
def kernel(logits, gumbel_u, interpret=False):
    hot = pl.pallas_call(
        _gumbel_argmax_kernel,
        in_specs=[
            pl.BlockSpec(memory_space=pl.ANY),
            pl.BlockSpec(memory_space=pl.ANY),
        ],
        out_specs=pl.BlockSpec((1, V), lambda: (0, 0)),
        out_shape=jax.ShapeDtypeStruct((1, V), jnp.float32),
        scratch_shapes=[
            pltpu.VMEM((NBUF, ROWS, V), jnp.float32),
            pltpu.VMEM((NBUF, ROWS, V), jnp.float32),
            pltpu.VMEM((B, 1), jnp.int32),
            pltpu.SemaphoreType.DMA((NBUF,)),
            pltpu.SemaphoreType.DMA((NBUF,)),
        ],
        interpret=interpret,
    )(logits, gumbel_u)

    row_is_zero = jax.lax.broadcasted_iota(jnp.int32, (B, V), 0) == 0
    return jnp.where(row_is_zero, hot, jnp.float32(0.0))
